# baseline (speedup 1.0000x reference)
import functools

import jax
import jax.numpy as jnp
from jax import lax
from jax.experimental import pallas as pl
from jax.experimental.pallas import tpu as pltpu
from jax.experimental.pallas import tpu_sc as plsc

N = 16384
K = 2
H = 1024
E = 16
NW = 32
C = 8
L = 16
NSL = H // L
NSPLIT = 2
NS = N // NSPLIT
RPW = NS // NW
NCHUNK = RPW // C
RTC = 512


def _sc_body(ex_hbm, idx0_hbm, idx1_hbm, s0_hbm, s1_hbm, pout_hbm,
             idx0_v, idx1_v, s0_v, s1_v,
             g0_v, g1_v, p_v, in_sems, out_sems):
  wid = lax.axis_index("s") * 2 + lax.axis_index("c")
  base = wid * RPW

  pltpu.sync_copy(idx0_hbm.at[pl.ds(base, RPW)], idx0_v)
  pltpu.sync_copy(idx1_hbm.at[pl.ds(base, RPW)], idx1_v)
  pltpu.sync_copy(s0_hbm.at[pl.ds(base, RPW)], s0_v)
  pltpu.sync_copy(s1_hbm.at[pl.ds(base, RPW)], s1_v)

  def issue_in(b2, c):
    pltpu.async_copy(ex_hbm.at[idx0_v.at[pl.ds(c * C, C)]], g0_v.at[b2],
                     in_sems[b2])
    pltpu.async_copy(ex_hbm.at[idx1_v.at[pl.ds(c * C, C)]], g1_v.at[b2],
                     in_sems[b2])

  def wait_in(b2, c):
    pltpu.make_async_copy(ex_hbm.at[idx0_v.at[pl.ds(c * C, C)]], g0_v.at[b2],
                          in_sems[b2]).wait()
    pltpu.make_async_copy(ex_hbm.at[idx1_v.at[pl.ds(c * C, C)]], g1_v.at[b2],
                          in_sems[b2]).wait()

  def issue_out(b4, c):
    rb = base + c * C
    pltpu.async_copy(p_v.at[b4], pout_hbm.at[pl.ds(rb, C)], out_sems[b4])

  def wait_out(b4, c):
    rb = base + c * C
    pltpu.make_async_copy(p_v.at[b4], pout_hbm.at[pl.ds(rb, C)],
                          out_sems[b4]).wait()

  def compute(b2, b4, c):
    s0rs, s1rs = [], []
    for r in range(C):
      rowvec = jnp.full((L,), c * C + r, jnp.int32)
      s0rs.append(plsc.load_gather(s0_v, [rowvec]))
      s1rs.append(plsc.load_gather(s1_v, [rowvec]))

    @plsc.parallel_loop(0, NSL, step=1, unroll=1)
    def _(h):
      off = h * L
      for r in range(C):
        g0 = g0_v[b2, r, pl.ds(off, L)]
        g1 = g1_v[b2, r, pl.ds(off, L)]
        p_v[b4, r, pl.ds(off, L)] = s0rs[r] * g0 + s1rs[r] * g1

  def do_chunk(b2, b4, c):
    wait_in(b2, c)

    @pl.when(c >= 4)
    def _():
      wait_out(b4, c - 4)

    compute(b2, b4, c)
    issue_out(b4, c)

    @pl.when(c + 2 < NCHUNK)
    def _():
      issue_in(b2, c + 2)

  issue_in(0, jnp.int32(0))
  issue_in(1, jnp.int32(1))

  def step(j, _):
    for k in range(4):
      do_chunk(k % 2, k, 4 * j + k)
    return 0

  lax.fori_loop(0, NCHUNK // 4, step, 0)
  for c in range(max(NCHUNK - 4, 0), NCHUNK):
    wait_out(c % 4, jnp.int32(c))


def _sc_partial(ex, idx0, idx1, s0, s1):
  mesh = plsc.VectorSubcoreMesh(core_axis_name="c", subcore_axis_name="s")
  f = pl.kernel(
      _sc_body,
      out_type=jax.ShapeDtypeStruct((NS, H), jnp.float32),
      mesh=mesh,
      compiler_params=pltpu.CompilerParams(needs_layout_passes=False),
      scratch_types=[
          pltpu.VMEM((RPW,), jnp.int32),
          pltpu.VMEM((RPW,), jnp.int32),
          pltpu.VMEM((RPW,), jnp.float32),
          pltpu.VMEM((RPW,), jnp.float32),
          pltpu.VMEM((2, C, H), jnp.float32),
          pltpu.VMEM((2, C, H), jnp.float32),
          pltpu.VMEM((4, C, H), jnp.float32),
          [pltpu.SemaphoreType.DMA] * 2,
          [pltpu.SemaphoreType.DMA] * 4,
      ],
  )
  return f(ex, idx0, idx1, s0, s1)


def _tc_body(x1_ref, x2_ref, p_ref, s0_ref, s1_ref, e0_ref, e1_ref,
             bias_ref, o_ref):
  ioe = lax.broadcasted_iota(jnp.int32, (RTC, E), 1)
  w = (jnp.where(ioe == e0_ref[...], s0_ref[...], 0.0) +
       jnp.where(ioe == e1_ref[...], s1_ref[...], 0.0))
  o_ref[...] = (x1_ref[...] + x2_ref[...] + p_ref[...] +
                jnp.dot(w, bias_ref[...], preferred_element_type=jnp.float32,
                        precision=lax.Precision.HIGHEST))


def _tc_finalize(x1, x2, partial, s0, s1, e0, e1, bias):
  nrow = x1.shape[0]
  row_spec = pl.BlockSpec((RTC, H), lambda i: (i, 0))
  col_spec = pl.BlockSpec((RTC, 1), lambda i: (i, 0))
  return pl.pallas_call(
      _tc_body,
      grid=(nrow // RTC,),
      in_specs=[row_spec, row_spec, row_spec,
                col_spec, col_spec, col_spec, col_spec,
                pl.BlockSpec((E, H), lambda i: (0, 0))],
      out_specs=row_spec,
      out_shape=jax.ShapeDtypeStruct((nrow, H), jnp.float32),
  )(x1, x2, partial, s0, s1, e0, e1, bias)


@jax.jit
def _run(ex, idx0, idx1, x1, x2, bias, s0, s1, e0, e1):
  partials = []
  for t in range(NSPLIT):
    r = slice(t * NS, (t + 1) * NS)
    partials.append(_sc_partial(ex, idx0[r], idx1[r], s0[r, 0], s1[r, 0]))
  outs = []
  for t in range(NSPLIT):
    r = slice(t * NS, (t + 1) * NS)
    outs.append(_tc_finalize(x1[r], x2[r], partials[t],
                             s0[r], s1[r], e0[r], e1[r], bias))
  return jnp.concatenate(outs, axis=0)


def kernel(expanded_x, expanded_row_idx, x1, x2, bias, scales, expert_idx,
           drop_pad_mode=0):
  idx0 = expanded_row_idx[:N]
  idx1 = expanded_row_idx[N:]
  s0 = scales[:, 0:1]
  s1 = scales[:, 1:2]
  e0 = expert_idx[:, 0:1]
  e1 = expert_idx[:, 1:2]
  return _run(expanded_x, idx0, idx1, x1, x2, bias, s0, s1, e0, e1)


# speedup vs baseline: 2.0658x; 2.0658x over previous
import jax
import jax.numpy as jnp
from jax import lax
from jax.experimental import pallas as pl
from jax.experimental.pallas import tpu as pltpu
from jax.experimental.pallas import tpu_sc as plsc

N = 16384
K = 2
H = 1024
E = 16
NW = 32
RPW = N // NW
C = 8
NCHUNK = RPW // C
L = 16
NSL = H // L


def _body(ex_hbm, idx0_hbm, idx1_hbm, x1_hbm, x2_hbm, biasf_hbm,
          s0_hbm, s1_hbm, e0_hbm, e1_hbm, out_hbm,
          idx0_v, idx1_v, s0_v, s1_v, e0_v, e1_v, bias_v,
          g0_v, g1_v, x1_v, x2_v, out_v,
          in_sem0, in_sem1, out_sem0, out_sem1):
  wid = lax.axis_index("s") * 2 + lax.axis_index("c")
  base = wid * RPW

  pltpu.sync_copy(idx0_hbm.at[pl.ds(base, RPW)], idx0_v)
  pltpu.sync_copy(idx1_hbm.at[pl.ds(base, RPW)], idx1_v)
  pltpu.sync_copy(s0_hbm.at[pl.ds(base, RPW)], s0_v)
  pltpu.sync_copy(s1_hbm.at[pl.ds(base, RPW)], s1_v)
  pltpu.sync_copy(e0_hbm.at[pl.ds(base, RPW)], e0_v)
  pltpu.sync_copy(e1_hbm.at[pl.ds(base, RPW)], e1_v)
  pltpu.sync_copy(biasf_hbm, bias_v)

  in_sems = (in_sem0, in_sem1)
  out_sems = (out_sem0, out_sem1)

  def issue_in(b, c):
    rb = base + c * C
    pltpu.async_copy(ex_hbm.at[idx0_v.at[pl.ds(c * C, C)]], g0_v.at[b],
                     in_sems[b])
    pltpu.async_copy(ex_hbm.at[idx1_v.at[pl.ds(c * C, C)]], g1_v.at[b],
                     in_sems[b])
    pltpu.async_copy(x1_hbm.at[pl.ds(rb, C)], x1_v.at[b], in_sems[b])
    pltpu.async_copy(x2_hbm.at[pl.ds(rb, C)], x2_v.at[b], in_sems[b])

  def wait_in(b, c):
    pltpu.make_async_copy(ex_hbm.at[idx0_v.at[pl.ds(c * C, C)]], g0_v.at[b],
                          in_sems[b]).wait()
    pltpu.make_async_copy(ex_hbm.at[idx1_v.at[pl.ds(c * C, C)]], g1_v.at[b],
                          in_sems[b]).wait()
    rb = base + c * C
    pltpu.make_async_copy(x1_hbm.at[pl.ds(rb, C)], x1_v.at[b],
                          in_sems[b]).wait()
    pltpu.make_async_copy(x2_hbm.at[pl.ds(rb, C)], x2_v.at[b],
                          in_sems[b]).wait()

  def issue_out(b, c):
    rb = base + c * C
    pltpu.async_copy(out_v.at[b], out_hbm.at[pl.ds(rb, C)], out_sems[b])

  def wait_out(b, c):
    rb = base + c * C
    pltpu.make_async_copy(out_v.at[b], out_hbm.at[pl.ds(rb, C)],
                          out_sems[b]).wait()

  lane = lax.iota(jnp.int32, L)

  def compute(b, c):
    s0rs, s1rs, eb0s, eb1s = [], [], [], []
    for r in range(C):
      rowvec = jnp.full((L,), c * C + r, jnp.int32)
      s0rs.append(plsc.load_gather(s0_v, [rowvec]))
      s1rs.append(plsc.load_gather(s1_v, [rowvec]))
      eb0s.append(plsc.load_gather(e0_v, [rowvec]) * H + lane)
      eb1s.append(plsc.load_gather(e1_v, [rowvec]) * H + lane)

    @plsc.parallel_loop(0, NSL, step=1, unroll=1)
    def _(h):
      off = h * L
      for r in range(C):
        g0 = g0_v[b, r, pl.ds(off, L)]
        g1 = g1_v[b, r, pl.ds(off, L)]
        a1 = x1_v[b, r, pl.ds(off, L)]
        a2 = x2_v[b, r, pl.ds(off, L)]
        b0 = plsc.load_gather(bias_v, [eb0s[r] + off])
        b1 = plsc.load_gather(bias_v, [eb1s[r] + off])
        out_v[b, r, pl.ds(off, L)] = (
            a1 + a2 + s0rs[r] * (g0 + b0) + s1rs[r] * (g1 + b1))

  issue_in(0, jnp.int32(0))

  def step(j, _):
    c0 = 2 * j
    c1 = c0 + 1
    issue_in(1, c1)
    wait_in(0, c0)

    @pl.when(c0 >= 2)
    def _():
      wait_out(0, c0 - 2)

    compute(0, c0)
    issue_out(0, c0)

    @pl.when(c0 + 2 < NCHUNK)
    def _():
      issue_in(0, c0 + 2)

    wait_in(1, c1)

    @pl.when(c1 >= 3)
    def _():
      wait_out(1, c1 - 2)

    compute(1, c1)
    issue_out(1, c1)
    return 0

  lax.fori_loop(0, NCHUNK // 2, step, 0)
  wait_out(0, jnp.int32(NCHUNK - 2))
  wait_out(1, jnp.int32(NCHUNK - 1))


@jax.jit
def _run(ex, idx0, idx1, x1, x2, biasf, s0, s1, e0, e1):
  mesh = plsc.VectorSubcoreMesh(core_axis_name="c", subcore_axis_name="s")
  f = pl.kernel(
      _body,
      out_type=jax.ShapeDtypeStruct((N, H), jnp.float32),
      mesh=mesh,
      compiler_params=pltpu.CompilerParams(needs_layout_passes=False),
      scratch_types=[
          pltpu.VMEM((RPW,), jnp.int32),
          pltpu.VMEM((RPW,), jnp.int32),
          pltpu.VMEM((RPW,), jnp.float32),
          pltpu.VMEM((RPW,), jnp.float32),
          pltpu.VMEM((RPW,), jnp.int32),
          pltpu.VMEM((RPW,), jnp.int32),
          pltpu.VMEM((E * H,), jnp.float32),
          pltpu.VMEM((2, C, H), jnp.float32),
          pltpu.VMEM((2, C, H), jnp.float32),
          pltpu.VMEM((2, C, H), jnp.float32),
          pltpu.VMEM((2, C, H), jnp.float32),
          pltpu.VMEM((2, C, H), jnp.float32),
          pltpu.SemaphoreType.DMA,
          pltpu.SemaphoreType.DMA,
          pltpu.SemaphoreType.DMA,
          pltpu.SemaphoreType.DMA,
      ],
  )
  return f(ex, idx0, idx1, x1, x2, biasf, s0, s1, e0, e1)


def kernel(expanded_x, expanded_row_idx, x1, x2, bias, scales, expert_idx,
           drop_pad_mode=0):
  idx0 = expanded_row_idx[:N]
  idx1 = expanded_row_idx[N:]
  biasf = bias.reshape(E * H)
  s0 = scales[:, 0]
  s1 = scales[:, 1]
  e0 = expert_idx[:, 0]
  e1 = expert_idx[:, 1]
  return _run(expanded_x, idx0, idx1, x1, x2, biasf, s0, s1, e0, e1)
